# baseline (speedup 1.0000x reference)
import functools

import jax
import jax.numpy as jnp
from jax import lax
from jax.experimental import pallas as pl
from jax.experimental.pallas import tpu as pltpu
from jax.experimental.pallas import tpu_sc as plsc

_BATCH = 16384
_DIM = 64
_NC = 2
_NS = 16
_NW = _NC * _NS
_NBIN = 248
_REG = 768
_SEL = 800
_LASTJ = 7812
_SENT = _BATCH


def _stage_a_body(tabT_hbm, tail_hbm, idx_hbm, et_hbm, bmap_hbm,
                  idx_c, sel_r, sel_b, colbuf, et_st, bmap_st, tmp_r, tmp_b):
    wid = lax.axis_index("s") * _NC + lax.axis_index("c")
    iota = lax.iota(jnp.int32, 16)
    lane0 = iota == 0

    neg1 = jnp.full((16,), -1, jnp.int32)
    sent = jnp.full((16,), _SENT, jnp.int32)
    for t in range(_SEL // 16):
        sel_r[pl.ds(t * 16, 16)] = neg1
    for rr in range(8):
        for t in range(8):
            bmap_st[rr, pl.ds(t * 16, 16)] = sent

    jlo = wid * _NBIN
    jhi = jlo + _NBIN

    def chunk_step(ch, pos):
        pltpu.sync_copy(idx_hbm.at[pl.ds(ch * 16, 16)], idx_c)

        def grp_step(g, pos):
            row = g >> 3
            c0 = (g & 7) * 16
            vec = idx_c[row, pl.ds(c0, 16)]
            binv = lax.shift_right_logical(vec, 7)
            m = (binv >= jlo) & (binv < jhi)
            bvec = iota + (ch * 2048 + g * 16)
            pc = plsc.cumsum(m.astype(jnp.int32))
            tgt = pos + pc - 1
            plsc.store_scatter(sel_r, [tgt], vec, mask=m)
            plsc.store_scatter(sel_b, [tgt], bvec, mask=m)
            return pos + pc[15]

        return lax.fori_loop(0, 128, grp_step, pos)

    lax.fori_loop(0, 8, chunk_step, 0)

    def bin_step(s, slot):
        j = jlo + s

        @pl.when(j < _LASTJ)
        def _():
            off = pl.multiple_of(j * 128, 128)
            pltpu.sync_copy(tabT_hbm.at[:, pl.ds(off, 128)], colbuf)

        @pl.when(j == _LASTJ)
        def _():
            pltpu.sync_copy(tail_hbm, colbuf)

        def scan_grp(t, slot):
            rvec = sel_r[pl.ds(t * 16, 16)]
            m = lax.shift_right_logical(rvec, 7) == j
            cnt = plsc.all_reduce_population_count(m)[0]

            def matched(slot):
                bvec = sel_b[pl.ds(t * 16, 16)]
                pc2 = plsc.cumsum(m.astype(jnp.int32))
                tgt2 = pc2 - 1
                plsc.store_scatter(tmp_r, [tgt2], rvec, mask=m)
                plsc.store_scatter(tmp_b, [tgt2], bvec, mask=m)

                def entry(e, slot):
                    rv = tmp_r[pl.ds(e, 16)]
                    bv = tmp_b[pl.ds(e, 16)]
                    col = rv[0] & 127
                    colv = jnp.full((16,), col, jnp.int32)
                    for q in range(4):
                        vals = plsc.load_gather(
                            colbuf, [iota + q * 16, colv]
                        )
                        et_st[slot, pl.ds(q * 16, 16)] = vals
                    plsc.store_scatter(
                        bmap_st,
                        [jnp.full((16,), slot >> 7, jnp.int32),
                         jnp.full((16,), slot & 127, jnp.int32)],
                        jnp.full((16,), bv[0], jnp.int32),
                        mask=lane0,
                    )
                    return slot + 1

                return lax.fori_loop(0, cnt, entry, slot)

            return lax.cond(cnt > 0, matched, lambda s_: s_, slot)

        return lax.fori_loop(0, _SEL // 16, scan_grp, slot)

    lax.fori_loop(0, _NBIN, bin_step, 0)

    for k in range(_REG // 128):
        pltpu.sync_copy(
            et_st.at[pl.ds(k * 128, 128)],
            et_hbm.at[pl.ds(wid * _REG + k * 128, 128)],
        )
    pltpu.sync_copy(bmap_st, bmap_hbm.at[pl.ds(wid * 8, 8)])


_stage_a = functools.partial(
    pl.kernel,
    mesh=plsc.VectorSubcoreMesh(core_axis_name="c", subcore_axis_name="s"),
    compiler_params=pltpu.CompilerParams(
        use_tc_tiling_on_sc=True, needs_layout_passes=False
    ),
    out_type=(
        jax.ShapeDtypeStruct((_NW * _REG, 128), jnp.float32),
        jax.ShapeDtypeStruct((_NW * 8, 128), jnp.int32),
    ),
    scratch_types=[
        pltpu.VMEM((16, 128), jnp.int32),
        pltpu.VMEM((_SEL,), jnp.int32),
        pltpu.VMEM((_SEL,), jnp.int32),
        pltpu.VMEM((_DIM, 128), jnp.float32),
        pltpu.VMEM((_REG, 128), jnp.float32),
        pltpu.VMEM((8, 128), jnp.int32),
        pltpu.VMEM((32,), jnp.int32),
        pltpu.VMEM((32,), jnp.int32),
    ],
)(_stage_a_body)


def _stage_b_body(et_hbm, bmap_hbm, out_hbm, chunk_v, row_v, bidx_v, sem):
    wid = lax.axis_index("s") * _NC + lax.axis_index("c")
    for k in range(_REG // 128):
        pltpu.sync_copy(
            et_hbm.at[pl.ds(wid * _REG + k * 128, 128)], chunk_v
        )
        pltpu.sync_copy(bmap_hbm.at[wid * 8 + k], bidx_v)

        def compact(i, _):
            for q in range(_DIM // 16):
                row_v[i, pl.ds(q * 16, 16)] = chunk_v[i, pl.ds(q * 16, 16)]
            return 0

        lax.fori_loop(0, 128, compact, 0)
        pltpu.async_copy(row_v, out_hbm.at[bidx_v], sem).wait()


_stage_b = functools.partial(
    pl.kernel,
    mesh=plsc.VectorSubcoreMesh(core_axis_name="c", subcore_axis_name="s"),
    compiler_params=pltpu.CompilerParams(
        use_tc_tiling_on_sc=False, needs_layout_passes=False
    ),
    out_type=jax.ShapeDtypeStruct((_BATCH + 8, _DIM), jnp.float32),
    scratch_types=[
        pltpu.VMEM((128, 128), jnp.float32),
        pltpu.VMEM((128, _DIM), jnp.float32),
        pltpu.VMEM((128,), jnp.int32),
        pltpu.SemaphoreType.DMA,
    ],
)(_stage_b_body)


def kernel(indices, table):
    tableT = table.T
    tail = jnp.pad(table[_LASTJ * 128:], ((0, 63), (0, 0))).T
    idx2d = indices.astype(jnp.int32).reshape(128, 128)
    et, bmap = _stage_a(tableT, tail, idx2d)
    out_pad = _stage_b(et, bmap)
    return out_pad[:_BATCH]


# speedup vs baseline: 1.4247x; 1.4247x over previous
import functools

import jax
import jax.numpy as jnp
from jax import lax
from jax.experimental import pallas as pl
from jax.experimental.pallas import tpu as pltpu
from jax.experimental.pallas import tpu_sc as plsc

_BATCH = 16384
_DIM = 64
_NC = 2
_NS = 16
_NW = _NC * _NS
_NBIN = 248
_REG = 768
_SEL = 800
_LASTJ = 7812
_SENT = _BATCH


def _stage_a_body(tabT_hbm, tail_hbm, idx_hbm, et_hbm, bmap_hbm,
                  idx_c, sel_r, sel_b, colbuf, et_st, bmap_st, tmp_r, tmp_b,
                  sem):
    wid = lax.axis_index("s") * _NC + lax.axis_index("c")
    iota = lax.iota(jnp.int32, 16)
    lane0 = iota == 0

    neg1 = jnp.full((16,), -1, jnp.int32)
    sent = jnp.full((16,), _SENT, jnp.int32)
    for t in range(_SEL // 16):
        sel_r[pl.ds(t * 16, 16)] = neg1
    for rr in range(8):
        for t in range(8):
            bmap_st[rr, pl.ds(t * 16, 16)] = sent

    jlo = wid * _NBIN
    jhi = jlo + _NBIN

    def chunk_step(ch, pos):
        pltpu.sync_copy(idx_hbm.at[pl.ds(ch * 16, 16)], idx_c)

        def grp_step(g, pos):
            row = g >> 3
            c0 = (g & 7) * 16
            vec = idx_c[row, pl.ds(c0, 16)]
            binv = lax.shift_right_logical(vec, 7)
            m = (binv >= jlo) & (binv < jhi)
            bvec = iota + (ch * 2048 + g * 16)
            pc = plsc.cumsum(m.astype(jnp.int32))
            tgt = pos + pc - 1
            plsc.store_scatter(sel_r, [tgt], vec, mask=m)
            plsc.store_scatter(sel_b, [tgt], bvec, mask=m)
            return pos + pc[15]

        return lax.fori_loop(0, 128, grp_step, pos)

    lax.fori_loop(0, 8, chunk_step, 0)

    def issue_fetch(s, p):
        j = jlo + s

        @pl.when(j < _LASTJ)
        def _():
            off = pl.multiple_of(j * 128, 128)
            pltpu.async_copy(tabT_hbm.at[:, pl.ds(off, 128)], colbuf.at[p], sem)

        @pl.when(j >= _LASTJ)
        def _():
            pltpu.async_copy(tail_hbm, colbuf.at[p], sem)

    issue_fetch(0, 0)

    def bin_step(s, slot):
        j = jlo + s

        @pl.when(s + 1 < _NBIN)
        def _():
            issue_fetch(s + 1, (s + 1) & 1)

        pltpu.make_async_copy(tail_hbm, colbuf.at[s & 1], sem).wait()
        cur = colbuf.at[s & 1]

        def scan_grp(t, slot):
            rvec = sel_r[pl.ds(t * 16, 16)]
            m = lax.shift_right_logical(rvec, 7) == j
            cnt = plsc.all_reduce_population_count(m)[0]

            def matched(slot):
                bvec = sel_b[pl.ds(t * 16, 16)]
                pc2 = plsc.cumsum(m.astype(jnp.int32))
                tgt2 = pc2 - 1
                plsc.store_scatter(tmp_r, [tgt2], rvec, mask=m)
                plsc.store_scatter(tmp_b, [tgt2], bvec, mask=m)

                def entry(e, slot):
                    rv = tmp_r[pl.ds(e, 16)]
                    bv = tmp_b[pl.ds(e, 16)]
                    col = rv[0] & 127
                    colv = jnp.full((16,), col, jnp.int32)
                    for q in range(4):
                        vals = plsc.load_gather(
                            cur, [iota + q * 16, colv]
                        )
                        et_st[slot, pl.ds(q * 16, 16)] = vals
                    plsc.store_scatter(
                        bmap_st,
                        [jnp.full((16,), slot >> 7, jnp.int32),
                         jnp.full((16,), slot & 127, jnp.int32)],
                        jnp.full((16,), bv[0], jnp.int32),
                        mask=lane0,
                    )
                    return slot + 1

                return lax.fori_loop(0, cnt, entry, slot)

            return lax.cond(cnt > 0, matched, lambda s_: s_, slot)

        return lax.fori_loop(0, _SEL // 16, scan_grp, slot)

    lax.fori_loop(0, _NBIN, bin_step, 0)

    for k in range(_REG // 128):
        pltpu.sync_copy(
            et_st.at[pl.ds(k * 128, 128)],
            et_hbm.at[pl.ds(wid * _REG + k * 128, 128)],
        )
    pltpu.sync_copy(bmap_st, bmap_hbm.at[pl.ds(wid * 8, 8)])


_stage_a = functools.partial(
    pl.kernel,
    mesh=plsc.VectorSubcoreMesh(core_axis_name="c", subcore_axis_name="s"),
    compiler_params=pltpu.CompilerParams(
        use_tc_tiling_on_sc=True, needs_layout_passes=False
    ),
    out_type=(
        jax.ShapeDtypeStruct((_NW * _REG, 128), jnp.float32),
        jax.ShapeDtypeStruct((_NW * 8, 128), jnp.int32),
    ),
    scratch_types=[
        pltpu.VMEM((16, 128), jnp.int32),
        pltpu.VMEM((_SEL,), jnp.int32),
        pltpu.VMEM((_SEL,), jnp.int32),
        pltpu.VMEM((2, _DIM, 128), jnp.float32),
        pltpu.VMEM((_REG, 128), jnp.float32),
        pltpu.VMEM((8, 128), jnp.int32),
        pltpu.VMEM((32,), jnp.int32),
        pltpu.VMEM((32,), jnp.int32),
        pltpu.SemaphoreType.DMA,
    ],
)(_stage_a_body)


def _stage_b_body(et_hbm, bmap_hbm, out_hbm, chunk_v, row_v, bidx_v, sem):
    wid = lax.axis_index("s") * _NC + lax.axis_index("c")
    for k in range(_REG // 128):
        pltpu.sync_copy(
            et_hbm.at[pl.ds(wid * _REG + k * 128, 128)], chunk_v
        )
        pltpu.sync_copy(bmap_hbm.at[wid * 8 + k], bidx_v)

        def compact(i, _):
            for q in range(_DIM // 16):
                row_v[i, pl.ds(q * 16, 16)] = chunk_v[i, pl.ds(q * 16, 16)]
            return 0

        lax.fori_loop(0, 128, compact, 0)
        pltpu.async_copy(row_v, out_hbm.at[bidx_v], sem).wait()


_stage_b = functools.partial(
    pl.kernel,
    mesh=plsc.VectorSubcoreMesh(core_axis_name="c", subcore_axis_name="s"),
    compiler_params=pltpu.CompilerParams(
        use_tc_tiling_on_sc=False, needs_layout_passes=False
    ),
    out_type=jax.ShapeDtypeStruct((_BATCH + 8, _DIM), jnp.float32),
    scratch_types=[
        pltpu.VMEM((128, 128), jnp.float32),
        pltpu.VMEM((128, _DIM), jnp.float32),
        pltpu.VMEM((128,), jnp.int32),
        pltpu.SemaphoreType.DMA,
    ],
)(_stage_b_body)


def kernel(indices, table):
    tableT = table.T
    tail = jnp.pad(table[_LASTJ * 128:], ((0, 63), (0, 0))).T
    idx2d = indices.astype(jnp.int32).reshape(128, 128)
    et, bmap = _stage_a(tableT, tail, idx2d)
    out_pad = _stage_b(et, bmap)
    return out_pad[:_BATCH]


# speedup vs baseline: 2.3283x; 1.6343x over previous
import functools

import jax
import jax.numpy as jnp
from jax import lax
from jax.experimental import pallas as pl
from jax.experimental.pallas import tpu as pltpu
from jax.experimental.pallas import tpu_sc as plsc

_BATCH = 16384
_DIM = 64
_NC = 2
_NS = 16
_NW = _NC * _NS
_NBIN = 248
_REG = 768
_SEL = 800
_LASTJ = 7812
_SENT = _BATCH


def _stage_a_body(tabT_hbm, tail_hbm, idx_hbm, et_hbm, bmap_hbm,
                  idx_c, sel_r, sel_b, colbuf, et_st, bmap_st, tmp_r, tmp_b,
                  sem):
    wid = lax.axis_index("s") * _NC + lax.axis_index("c")
    iota = lax.iota(jnp.int32, 16)
    lane0 = iota == 0

    neg1 = jnp.full((16,), -1, jnp.int32)
    for t in range(_SEL // 16):
        sel_r[pl.ds(t * 16, 16)] = neg1
    for rr in range(8):
        for t in range(8):
            bmap_st[rr, pl.ds(t * 16, 16)] = _SENT + t * 16 + iota

    jlo = wid * _NBIN
    jhi = jlo + _NBIN

    def chunk_step(ch, pos):
        pltpu.sync_copy(idx_hbm.at[pl.ds(ch * 16, 16)], idx_c)

        def grp_step(g, pos):
            row = g >> 3
            c0 = (g & 7) * 16
            vec = idx_c[row, pl.ds(c0, 16)]
            binv = lax.shift_right_logical(vec, 7)
            m = (binv >= jlo) & (binv < jhi)
            bvec = iota + (ch * 2048 + g * 16)
            pc = plsc.cumsum(m.astype(jnp.int32))
            tgt = pos + pc - 1
            plsc.store_scatter(sel_r, [tgt], vec, mask=m)
            plsc.store_scatter(sel_b, [tgt], bvec, mask=m)
            return pos + pc[15]

        return lax.fori_loop(0, 128, grp_step, pos)

    nsel = lax.fori_loop(0, 8, chunk_step, 0)
    ngrp = (nsel + 15) >> 4

    def issue_fetch(s, p):
        j = jlo + s

        @pl.when(j < _LASTJ)
        def _():
            off = pl.multiple_of(j * 128, 128)
            pltpu.async_copy(tabT_hbm.at[:, pl.ds(off, 128)], colbuf.at[p], sem)

        @pl.when(j >= _LASTJ)
        def _():
            pltpu.async_copy(tail_hbm, colbuf.at[p], sem)

    issue_fetch(0, 0)

    def bin_step(s, slot):
        j = jlo + s

        @pl.when(s + 1 < _NBIN)
        def _():
            issue_fetch(s + 1, (s + 1) & 1)

        pltpu.make_async_copy(tail_hbm, colbuf.at[s & 1], sem).wait()
        cur = colbuf.at[s & 1]

        def scan_grp(t, slot):
            rvec = sel_r[pl.ds(t * 16, 16)]
            m = lax.shift_right_logical(rvec, 7) == j
            cnt = plsc.all_reduce_population_count(m)[0]

            def matched(slot):
                bvec = sel_b[pl.ds(t * 16, 16)]
                pc2 = plsc.cumsum(m.astype(jnp.int32))
                tgt2 = pc2 - 1
                plsc.store_scatter(tmp_r, [tgt2], rvec, mask=m)
                plsc.store_scatter(tmp_b, [tgt2], bvec, mask=m)

                def entry(e, slot):
                    rv = tmp_r[pl.ds(e, 16)]
                    bv = tmp_b[pl.ds(e, 16)]
                    col = rv[0] & 127
                    colv = jnp.full((16,), col, jnp.int32)
                    for q in range(4):
                        vals = plsc.load_gather(
                            cur, [iota + q * 16, colv]
                        )
                        et_st[slot, pl.ds(q * 16, 16)] = vals
                    plsc.store_scatter(
                        bmap_st,
                        [jnp.full((16,), slot >> 7, jnp.int32),
                         jnp.full((16,), slot & 127, jnp.int32)],
                        jnp.full((16,), bv[0], jnp.int32),
                        mask=lane0,
                    )
                    return slot + 1

                return lax.fori_loop(0, cnt, entry, slot)

            return lax.cond(cnt > 0, matched, lambda s_: s_, slot)

        return lax.fori_loop(0, ngrp, scan_grp, slot)

    lax.fori_loop(0, _NBIN, bin_step, 0)

    for k in range(_REG // 128):
        pltpu.sync_copy(
            et_st.at[pl.ds(k * 128, 128)],
            et_hbm.at[pl.ds(wid * _REG + k * 128, 128)],
        )
    pltpu.sync_copy(bmap_st, bmap_hbm.at[pl.ds(wid * 8, 8)])


_stage_a = functools.partial(
    pl.kernel,
    mesh=plsc.VectorSubcoreMesh(core_axis_name="c", subcore_axis_name="s"),
    compiler_params=pltpu.CompilerParams(
        use_tc_tiling_on_sc=True, needs_layout_passes=False
    ),
    out_type=(
        jax.ShapeDtypeStruct((_NW * _REG, 128), jnp.float32),
        jax.ShapeDtypeStruct((_NW * 8, 128), jnp.int32),
    ),
    scratch_types=[
        pltpu.VMEM((16, 128), jnp.int32),
        pltpu.VMEM((_SEL,), jnp.int32),
        pltpu.VMEM((_SEL,), jnp.int32),
        pltpu.VMEM((2, _DIM, 128), jnp.float32),
        pltpu.VMEM((_REG, 128), jnp.float32),
        pltpu.VMEM((8, 128), jnp.int32),
        pltpu.VMEM((32,), jnp.int32),
        pltpu.VMEM((32,), jnp.int32),
        pltpu.SemaphoreType.DMA,
    ],
)(_stage_a_body)


def _stage_b_body(et_hbm, bmap_hbm, out_hbm, chunk_v, row_v, bidx_v, sem):
    wid = lax.axis_index("s") * _NC + lax.axis_index("c")
    for k in range(_REG // 128):
        pltpu.sync_copy(
            et_hbm.at[pl.ds(wid * _REG + k * 128, 128)], chunk_v
        )
        pltpu.sync_copy(bmap_hbm.at[wid * 8 + k], bidx_v)

        def compact(i, _):
            for q in range(_DIM // 16):
                row_v[i, pl.ds(q * 16, 16)] = chunk_v[i, pl.ds(q * 16, 16)]
            return 0

        lax.fori_loop(0, 128, compact, 0)
        pltpu.async_copy(row_v, out_hbm.at[bidx_v], sem).wait()


_stage_b = functools.partial(
    pl.kernel,
    mesh=plsc.VectorSubcoreMesh(core_axis_name="c", subcore_axis_name="s"),
    compiler_params=pltpu.CompilerParams(
        use_tc_tiling_on_sc=False, needs_layout_passes=False
    ),
    out_type=jax.ShapeDtypeStruct((_BATCH + 128, _DIM), jnp.float32),
    scratch_types=[
        pltpu.VMEM((128, 128), jnp.float32),
        pltpu.VMEM((128, _DIM), jnp.float32),
        pltpu.VMEM((128,), jnp.int32),
        pltpu.SemaphoreType.DMA,
    ],
)(_stage_b_body)


def kernel(indices, table):
    tableT = table.T
    tail = jnp.pad(table[_LASTJ * 128:], ((0, 63), (0, 0))).T
    idx2d = indices.astype(jnp.int32).reshape(128, 128)
    et, bmap = _stage_a(tableT, tail, idx2d)
    out_pad = _stage_b(et, bmap)
    return out_pad[:_BATCH]


# speedup vs baseline: 3.9499x; 1.6965x over previous
import functools

import jax
import jax.numpy as jnp
from jax import lax
from jax.experimental import pallas as pl
from jax.experimental.pallas import tpu as pltpu
from jax.experimental.pallas import tpu_sc as plsc

_BATCH = 16384
_DIM = 64
_NC = 2
_NS = 16
_NW = _NC * _NS
_SBW = 512
_NSB = 62
_REG = 768
_SEL = 800
_LASTSB = 1953
_SENT = _BATCH


def _stage_a_body(tabT_hbm, tail_hbm, idx_hbm, et_hbm, bmap_hbm,
                  idx_c, sel_r, sel_b, colbuf, et_st, bmap_st, tmp_r, tmp_b,
                  sem):
    wid = lax.axis_index("s") * _NC + lax.axis_index("c")
    iota = lax.iota(jnp.int32, 16)
    lane0 = iota == 0

    neg1 = jnp.full((16,), -1, jnp.int32)
    for t in range(_SEL // 16):
        sel_r[pl.ds(t * 16, 16)] = neg1
    for rr in range(8):
        for t in range(8):
            bmap_st[rr, pl.ds(t * 16, 16)] = _SENT + t * 16 + iota

    glo = wid * _NSB

    def issue_fetch(s, p):
        g = glo + s

        @pl.when(g < _LASTSB)
        def _():
            off = pl.multiple_of(g * _SBW, 128)
            pltpu.async_copy(
                tabT_hbm.at[:, pl.ds(off, _SBW)], colbuf.at[p], sem
            )

        @pl.when(g >= _LASTSB)
        def _():
            for u in range(4):
                pltpu.async_copy(
                    tail_hbm, colbuf.at[p, :, pl.ds(u * 128, 128)], sem
                )

    issue_fetch(0, 0)
    issue_fetch(1, 1)

    def chunk_step(ch, pos):
        pltpu.sync_copy(idx_hbm.at[pl.ds(ch * 16, 16)], idx_c)

        def grp_step(g, pos):
            row = g >> 3
            c0 = (g & 7) * 16
            vec = idx_c[row, pl.ds(c0, 16)]
            blk = lax.shift_right_logical(vec, 9)
            m = (blk >= glo) & (blk < glo + _NSB)
            bvec = iota + (ch * 2048 + g * 16)
            pc = plsc.cumsum(m.astype(jnp.int32))
            tgt = pos + pc - 1
            plsc.store_scatter(sel_r, [tgt], vec, mask=m)
            plsc.store_scatter(sel_b, [tgt], bvec, mask=m)
            return pos + pc[15]

        return lax.fori_loop(0, 128, grp_step, pos)

    nsel = lax.fori_loop(0, 8, chunk_step, 0)
    ngrp = (nsel + 15) >> 4

    def blk_step(s, carry):
        slot, flushed = carry
        g = glo + s

        pltpu.make_async_copy(
            tabT_hbm.at[:, pl.ds(0, _SBW)], colbuf.at[s & 1], sem
        ).wait()

        cur = colbuf.at[s & 1]

        def scan_grp(t, slot):
            rvec = sel_r[pl.ds(t * 16, 16)]
            m = lax.shift_right_logical(rvec, 9) == g
            cnt = plsc.all_reduce_population_count(m)[0]

            def matched(slot):
                bvec = sel_b[pl.ds(t * 16, 16)]
                pc2 = plsc.cumsum(m.astype(jnp.int32))
                tgt2 = pc2 - 1
                plsc.store_scatter(tmp_r, [tgt2], rvec, mask=m)
                plsc.store_scatter(tmp_b, [tgt2], bvec, mask=m)

                def entry(e, slot):
                    rv = tmp_r[pl.ds(e, 16)]
                    bv = tmp_b[pl.ds(e, 16)]
                    col = rv[0] & (_SBW - 1)
                    colv = jnp.full((16,), col, jnp.int32)
                    srow = slot & 255
                    for q in range(4):
                        vals = plsc.load_gather(cur, [iota + q * 16, colv])
                        et_st[srow, pl.ds(q * 16, 16)] = vals
                    plsc.store_scatter(
                        bmap_st,
                        [jnp.full((16,), slot >> 7, jnp.int32),
                         jnp.full((16,), slot & 127, jnp.int32)],
                        jnp.full((16,), bv[0], jnp.int32),
                        mask=lane0,
                    )
                    return slot + 1

                return lax.fori_loop(0, cnt, entry, slot)

            return lax.cond(cnt > 0, matched, lambda s_: s_, slot)

        slot = lax.fori_loop(0, ngrp, scan_grp, slot)

        @pl.when(s + 2 < _NSB)
        def _():
            issue_fetch(s + 2, s & 1)

        do = (slot - flushed) >= 128

        @pl.when(do)
        def _():
            pltpu.sync_copy(
                et_st.at[pl.ds(pl.multiple_of(flushed & 255, 128), 128)],
                et_hbm.at[pl.ds(pl.multiple_of(wid * _REG + flushed, 128), 128)],
            )

        return slot, flushed + 128 * do.astype(jnp.int32)

    slot, flushed = lax.fori_loop(0, _NSB, blk_step, (0, 0))

    for _ in range(2):
        do = slot > flushed

        @pl.when(do)
        def _():
            pltpu.sync_copy(
                et_st.at[pl.ds(pl.multiple_of(flushed & 255, 128), 128)],
                et_hbm.at[pl.ds(pl.multiple_of(wid * _REG + flushed, 128), 128)],
            )

        flushed = flushed + 128 * do.astype(jnp.int32)

    pltpu.sync_copy(bmap_st, bmap_hbm.at[pl.ds(wid * 8, 8)])


_stage_a = functools.partial(
    pl.kernel,
    mesh=plsc.VectorSubcoreMesh(core_axis_name="c", subcore_axis_name="s"),
    compiler_params=pltpu.CompilerParams(
        use_tc_tiling_on_sc=True, needs_layout_passes=False
    ),
    out_type=(
        jax.ShapeDtypeStruct((_NW * _REG, 128), jnp.float32),
        jax.ShapeDtypeStruct((_NW * 8, 128), jnp.int32),
    ),
    scratch_types=[
        pltpu.VMEM((16, 128), jnp.int32),
        pltpu.VMEM((_SEL,), jnp.int32),
        pltpu.VMEM((_SEL,), jnp.int32),
        pltpu.VMEM((2, _DIM, _SBW), jnp.float32),
        pltpu.VMEM((256, 128), jnp.float32),
        pltpu.VMEM((8, 128), jnp.int32),
        pltpu.VMEM((32,), jnp.int32),
        pltpu.VMEM((32,), jnp.int32),
        pltpu.SemaphoreType.DMA,
    ],
)(_stage_a_body)


def _stage_b_body(et_hbm, bmap_hbm, out_hbm, chunk_v, row_v, bidx_v, sem):
    wid = lax.axis_index("s") * _NC + lax.axis_index("c")
    for k in range(_REG // 128):
        pltpu.sync_copy(
            et_hbm.at[pl.ds(wid * _REG + k * 128, 128)], chunk_v
        )
        pltpu.sync_copy(bmap_hbm.at[wid * 8 + k], bidx_v)

        def compact(i, _):
            for q in range(_DIM // 16):
                row_v[i, pl.ds(q * 16, 16)] = chunk_v[i, pl.ds(q * 16, 16)]
            return 0

        lax.fori_loop(0, 128, compact, 0)
        pltpu.async_copy(row_v, out_hbm.at[bidx_v], sem).wait()


_stage_b = functools.partial(
    pl.kernel,
    mesh=plsc.VectorSubcoreMesh(core_axis_name="c", subcore_axis_name="s"),
    compiler_params=pltpu.CompilerParams(
        use_tc_tiling_on_sc=False, needs_layout_passes=False
    ),
    out_type=jax.ShapeDtypeStruct((_BATCH + 128, _DIM), jnp.float32),
    scratch_types=[
        pltpu.VMEM((128, 128), jnp.float32),
        pltpu.VMEM((128, _DIM), jnp.float32),
        pltpu.VMEM((128,), jnp.int32),
        pltpu.SemaphoreType.DMA,
    ],
)(_stage_b_body)


def kernel(indices, table):
    tableT = table.T
    tail = jnp.pad(table[_LASTSB * _SBW:], ((0, 63), (0, 0))).T
    idx2d = indices.astype(jnp.int32).reshape(128, 128)
    et, bmap = _stage_a(tableT, tail, idx2d)
    out_pad = _stage_b(et, bmap)
    return out_pad[:_BATCH]


# speedup vs baseline: 3.9940x; 1.0112x over previous
import functools

import jax
import jax.numpy as jnp
from jax import lax
from jax.experimental import pallas as pl
from jax.experimental.pallas import tpu as pltpu
from jax.experimental.pallas import tpu_sc as plsc

_BATCH = 16384
_DIM = 64
_NC = 2
_NS = 16
_NW = _NC * _NS
_SBW = 512
_NSB = 62
_REG = 768
_SEL = 800
_LASTSB = 1953
_SENT = _BATCH


def _stage_a_body(tabT_hbm, tail_hbm, idx_hbm, et_hbm, bmap_hbm,
                  idx_c, sel_r, sel_b, colbuf, et_st, bmap_st, tmp_r, tmp_b,
                  sem):
    wid = lax.axis_index("s") * _NC + lax.axis_index("c")
    iota = lax.iota(jnp.int32, 16)
    lane0 = iota == 0

    neg1 = jnp.full((16,), -1, jnp.int32)
    for t in range(_SEL // 16):
        sel_r[pl.ds(t * 16, 16)] = neg1
    for rr in range(8):
        for t in range(8):
            bmap_st[rr, pl.ds(t * 16, 16)] = _SENT + t * 16 + iota

    glo = wid * _NSB

    def issue_fetch(s, p):
        g = glo + s

        @pl.when(g < _LASTSB)
        def _():
            off = pl.multiple_of(g * _SBW, 128)
            pltpu.async_copy(
                tabT_hbm.at[:, pl.ds(off, _SBW)], colbuf.at[p], sem
            )

        @pl.when(g >= _LASTSB)
        def _():
            for u in range(4):
                pltpu.async_copy(
                    tail_hbm, colbuf.at[p, :, pl.ds(u * 128, 128)], sem
                )

    issue_fetch(0, 0)
    issue_fetch(1, 1)

    pltpu.sync_copy(idx_hbm, idx_c)

    def grp_step(g, pos):
        row = g >> 3
        c0 = (g & 7) * 16
        vec = idx_c[row, pl.ds(c0, 16)]
        blk = lax.shift_right_logical(vec, 9)
        m = (blk >= glo) & (blk < glo + _NSB)
        bvec = iota + g * 16
        pc = plsc.cumsum(m.astype(jnp.int32))
        tgt = pos + pc - 1
        plsc.store_scatter(sel_r, [tgt], vec, mask=m)
        plsc.store_scatter(sel_b, [tgt], bvec, mask=m)
        return pos + pc[15]

    nsel = lax.fori_loop(0, 1024, grp_step, 0)
    ngrp = (nsel + 15) >> 4

    def blk_step(s, carry):
        slot, flushed = carry
        g = glo + s

        pltpu.make_async_copy(
            tabT_hbm.at[:, pl.ds(0, _SBW)], colbuf.at[s & 1], sem
        ).wait()

        cur = colbuf.at[s & 1]

        def scan_grp(t, slot):
            rvec = sel_r[pl.ds(t * 16, 16)]
            m = lax.shift_right_logical(rvec, 9) == g
            cnt = plsc.all_reduce_population_count(m)[0]

            def matched(slot):
                bvec = sel_b[pl.ds(t * 16, 16)]
                pc2 = plsc.cumsum(m.astype(jnp.int32))
                tgt2 = pc2 - 1
                plsc.store_scatter(tmp_r, [tgt2], rvec, mask=m)
                plsc.store_scatter(tmp_b, [tgt2], bvec, mask=m)

                def entry(e, slot):
                    rv = tmp_r[pl.ds(e, 16)]
                    bv = tmp_b[pl.ds(e, 16)]
                    col = rv[0] & (_SBW - 1)
                    colv = jnp.full((16,), col, jnp.int32)
                    srow = slot & 255
                    for q in range(4):
                        vals = plsc.load_gather(cur, [iota + q * 16, colv])
                        et_st[srow, pl.ds(q * 16, 16)] = vals
                    plsc.store_scatter(
                        bmap_st,
                        [jnp.full((16,), slot >> 7, jnp.int32),
                         jnp.full((16,), slot & 127, jnp.int32)],
                        jnp.full((16,), bv[0], jnp.int32),
                        mask=lane0,
                    )
                    return slot + 1

                return lax.fori_loop(0, cnt, entry, slot)

            return lax.cond(cnt > 0, matched, lambda s_: s_, slot)

        slot = lax.fori_loop(0, ngrp, scan_grp, slot)

        @pl.when(s + 2 < _NSB)
        def _():
            issue_fetch(s + 2, s & 1)

        do = (slot - flushed) >= 128

        @pl.when(do)
        def _():
            pltpu.sync_copy(
                et_st.at[pl.ds(pl.multiple_of(flushed & 255, 128), 128)],
                et_hbm.at[pl.ds(pl.multiple_of(wid * _REG + flushed, 128), 128)],
            )

        return slot, flushed + 128 * do.astype(jnp.int32)

    slot, flushed = lax.fori_loop(0, _NSB, blk_step, (0, 0))

    for _ in range(2):
        do = slot > flushed

        @pl.when(do)
        def _():
            pltpu.sync_copy(
                et_st.at[pl.ds(pl.multiple_of(flushed & 255, 128), 128)],
                et_hbm.at[pl.ds(pl.multiple_of(wid * _REG + flushed, 128), 128)],
            )

        flushed = flushed + 128 * do.astype(jnp.int32)

    pltpu.sync_copy(bmap_st, bmap_hbm.at[pl.ds(wid * 8, 8)])


_stage_a = functools.partial(
    pl.kernel,
    mesh=plsc.VectorSubcoreMesh(core_axis_name="c", subcore_axis_name="s"),
    compiler_params=pltpu.CompilerParams(
        use_tc_tiling_on_sc=True, needs_layout_passes=False
    ),
    out_type=(
        jax.ShapeDtypeStruct((_NW * _REG, 128), jnp.float32),
        jax.ShapeDtypeStruct((_NW * 8, 128), jnp.int32),
    ),
    scratch_types=[
        pltpu.VMEM((128, 128), jnp.int32),
        pltpu.VMEM((_SEL,), jnp.int32),
        pltpu.VMEM((_SEL,), jnp.int32),
        pltpu.VMEM((2, _DIM, _SBW), jnp.float32),
        pltpu.VMEM((256, 128), jnp.float32),
        pltpu.VMEM((8, 128), jnp.int32),
        pltpu.VMEM((32,), jnp.int32),
        pltpu.VMEM((32,), jnp.int32),
        pltpu.SemaphoreType.DMA,
    ],
)(_stage_a_body)


def _stage_b_body(et_hbm, bmap_hbm, out_hbm, chunk_v, row_v, bidx_v, sem):
    wid = lax.axis_index("s") * _NC + lax.axis_index("c")
    for k in range(_REG // 128):
        pltpu.sync_copy(
            et_hbm.at[pl.ds(wid * _REG + k * 128, 128)], chunk_v
        )
        pltpu.sync_copy(bmap_hbm.at[wid * 8 + k], bidx_v)

        def compact(i, _):
            for q in range(_DIM // 16):
                row_v[i, pl.ds(q * 16, 16)] = chunk_v[i, pl.ds(q * 16, 16)]
            return 0

        lax.fori_loop(0, 128, compact, 0)
        pltpu.async_copy(row_v, out_hbm.at[bidx_v], sem).wait()


_stage_b = functools.partial(
    pl.kernel,
    mesh=plsc.VectorSubcoreMesh(core_axis_name="c", subcore_axis_name="s"),
    compiler_params=pltpu.CompilerParams(
        use_tc_tiling_on_sc=False, needs_layout_passes=False
    ),
    out_type=jax.ShapeDtypeStruct((_BATCH + 128, _DIM), jnp.float32),
    scratch_types=[
        pltpu.VMEM((128, 128), jnp.float32),
        pltpu.VMEM((128, _DIM), jnp.float32),
        pltpu.VMEM((128,), jnp.int32),
        pltpu.SemaphoreType.DMA,
    ],
)(_stage_b_body)


def kernel(indices, table):
    tableT = table.T
    tail = jnp.pad(table[_LASTSB * _SBW:], ((0, 63), (0, 0))).T
    idx2d = indices.astype(jnp.int32).reshape(128, 128)
    et, bmap = _stage_a(tableT, tail, idx2d)
    out_pad = _stage_b(et, bmap)
    return out_pad[:_BATCH]


# speedup vs baseline: 4.1818x; 1.0470x over previous
import functools

import jax
import jax.numpy as jnp
from jax import lax
from jax.experimental import pallas as pl
from jax.experimental.pallas import tpu as pltpu
from jax.experimental.pallas import tpu_sc as plsc

_BATCH = 16384
_DIM = 64
_NC = 2
_NS = 16
_NW = _NC * _NS
_SBW = 512
_NSB = 62
_REG = 768
_SEL = 800
_LASTSB = 1953
_SENT = _BATCH


def _stage_a_body(tabT_hbm, tail_hbm, idx_hbm, et_hbm, bmap_hbm,
                  idx_c, sel_r, sel_b, colbuf, et_st, bmap_st, tmp_r, tmp_b,
                  sem):
    wid = lax.axis_index("s") * _NC + lax.axis_index("c")
    iota = lax.iota(jnp.int32, 16)
    lane0 = iota == 0

    neg1 = jnp.full((16,), -1, jnp.int32)
    for t in range(_SEL // 16):
        sel_r[pl.ds(t * 16, 16)] = neg1
    for rr in range(8):
        for t in range(8):
            bmap_st[rr, pl.ds(t * 16, 16)] = _SENT + t * 16 + iota

    glo = wid * _NSB

    def issue_fetch(s, p):
        g = glo + s

        @pl.when(g < _LASTSB)
        def _():
            off = pl.multiple_of(g * _SBW, 128)
            pltpu.async_copy(
                tabT_hbm.at[:, pl.ds(off, _SBW)], colbuf.at[p], sem
            )

        @pl.when(g >= _LASTSB)
        def _():
            for u in range(4):
                pltpu.async_copy(
                    tail_hbm, colbuf.at[p, :, pl.ds(u * 128, 128)], sem
                )

    issue_fetch(0, 0)
    issue_fetch(1, 1)

    pltpu.sync_copy(idx_hbm, idx_c)

    def grp_step(g, pos):
        row = g >> 3
        c0 = (g & 7) * 16
        vec = idx_c[row, pl.ds(c0, 16)]
        blk = lax.shift_right_logical(vec, 9)
        m = (blk >= glo) & (blk < glo + _NSB)
        bvec = iota + g * 16
        pc = plsc.cumsum(m.astype(jnp.int32))
        tgt = pos + pc - 1
        plsc.store_scatter(sel_r, [tgt], vec, mask=m)
        plsc.store_scatter(sel_b, [tgt], bvec, mask=m)
        return pos + pc[15]

    nsel = lax.fori_loop(0, 1024, grp_step, 0)
    ngrp = (nsel + 15) >> 4

    def blk_step(s, carry):
        slot, flushed = carry
        g = glo + s

        pltpu.make_async_copy(
            tabT_hbm.at[:, pl.ds(0, _SBW)], colbuf.at[s & 1], sem
        ).wait()

        cur = colbuf.at[s & 1]

        def scan_grp(t, slot):
            rvec = sel_r[pl.ds(t * 16, 16)]
            m = lax.shift_right_logical(rvec, 9) == g
            cnt = plsc.all_reduce_population_count(m)[0]

            def matched(slot):
                bvec = sel_b[pl.ds(t * 16, 16)]
                pc2 = plsc.cumsum(m.astype(jnp.int32))
                tgt2 = pc2 - 1
                plsc.store_scatter(tmp_r, [tgt2], rvec, mask=m)
                plsc.store_scatter(tmp_b, [tgt2], bvec, mask=m)

                def entry(e, slot):
                    rv = tmp_r[pl.ds(e, 16)]
                    bv = tmp_b[pl.ds(e, 16)]
                    col = rv[0] & (_SBW - 1)
                    colv = jnp.full((16,), col, jnp.int32)
                    srow = slot & 255
                    for q in range(4):
                        vals = plsc.load_gather(cur, [iota + q * 16, colv])
                        et_st[srow, pl.ds(q * 16, 16)] = vals
                    plsc.store_scatter(
                        bmap_st,
                        [jnp.full((16,), slot >> 7, jnp.int32),
                         jnp.full((16,), slot & 127, jnp.int32)],
                        jnp.full((16,), bv[0], jnp.int32),
                        mask=lane0,
                    )
                    return slot + 1

                return lax.fori_loop(0, cnt, entry, slot)

            return lax.cond(cnt > 0, matched, lambda s_: s_, slot)

        slot = lax.fori_loop(0, ngrp, scan_grp, slot)

        @pl.when(s + 2 < _NSB)
        def _():
            issue_fetch(s + 2, s & 1)

        do = (slot - flushed) >= 128

        @pl.when(do)
        def _():
            pltpu.sync_copy(
                et_st.at[pl.ds(pl.multiple_of(flushed & 255, 128), 128)],
                et_hbm.at[pl.ds(pl.multiple_of(wid * _REG + flushed, 128), 128)],
            )

        return slot, flushed + 128 * do.astype(jnp.int32)

    slot, flushed = lax.fori_loop(0, _NSB, blk_step, (0, 0))

    for _ in range(2):
        do = slot > flushed

        @pl.when(do)
        def _():
            pltpu.sync_copy(
                et_st.at[pl.ds(pl.multiple_of(flushed & 255, 128), 128)],
                et_hbm.at[pl.ds(pl.multiple_of(wid * _REG + flushed, 128), 128)],
            )

        flushed = flushed + 128 * do.astype(jnp.int32)

    pltpu.sync_copy(bmap_st, bmap_hbm.at[pl.ds(wid * 8, 8)])


_stage_a = functools.partial(
    pl.kernel,
    mesh=plsc.VectorSubcoreMesh(core_axis_name="c", subcore_axis_name="s"),
    compiler_params=pltpu.CompilerParams(
        use_tc_tiling_on_sc=True, needs_layout_passes=False
    ),
    out_type=(
        jax.ShapeDtypeStruct((_NW * _REG, 128), jnp.float32),
        jax.ShapeDtypeStruct((_NW * 8, 128), jnp.int32),
    ),
    scratch_types=[
        pltpu.VMEM((128, 128), jnp.int32),
        pltpu.VMEM((_SEL,), jnp.int32),
        pltpu.VMEM((_SEL,), jnp.int32),
        pltpu.VMEM((2, _DIM, _SBW), jnp.float32),
        pltpu.VMEM((256, 128), jnp.float32),
        pltpu.VMEM((8, 128), jnp.int32),
        pltpu.VMEM((32,), jnp.int32),
        pltpu.VMEM((32,), jnp.int32),
        pltpu.SemaphoreType.DMA,
    ],
)(_stage_a_body)


def _stage_b_body(et_hbm, bmap_hbm, out_hbm, chunk_v, row_v, bidx_v, sem,
                  sem2):
    wid = lax.axis_index("s") * _NC + lax.axis_index("c")
    nchunk = _REG // 128
    pltpu.sync_copy(bmap_hbm.at[pl.ds(wid * 8, 8)], bidx_v)
    pltpu.async_copy(et_hbm.at[pl.ds(wid * _REG, 128)], chunk_v.at[0], sem)
    pending = None
    for k in range(nchunk):
        pltpu.make_async_copy(
            et_hbm.at[pl.ds(0, 128)], chunk_v.at[k & 1], sem
        ).wait()
        if k + 1 < nchunk:
            pltpu.async_copy(
                et_hbm.at[pl.ds(wid * _REG + (k + 1) * 128, 128)],
                chunk_v.at[(k + 1) & 1],
                sem,
            )
        if pending is not None:
            pending.wait()

        def compact(i, _, k=k):
            for q in range(_DIM // 16):
                row_v[k & 1, i, pl.ds(q * 16, 16)] = (
                    chunk_v[k & 1, i, pl.ds(q * 16, 16)]
                )
            return 0

        lax.fori_loop(0, 128, compact, 0)
        pending = pltpu.async_copy(
            row_v.at[k & 1], out_hbm.at[bidx_v.at[k]], sem2
        )
    pending.wait()


_stage_b = functools.partial(
    pl.kernel,
    mesh=plsc.VectorSubcoreMesh(core_axis_name="c", subcore_axis_name="s"),
    compiler_params=pltpu.CompilerParams(
        use_tc_tiling_on_sc=False, needs_layout_passes=False
    ),
    out_type=jax.ShapeDtypeStruct((_BATCH + 128, _DIM), jnp.float32),
    scratch_types=[
        pltpu.VMEM((2, 128, 128), jnp.float32),
        pltpu.VMEM((2, 128, _DIM), jnp.float32),
        pltpu.VMEM((8, 128), jnp.int32),
        pltpu.SemaphoreType.DMA,
        pltpu.SemaphoreType.DMA,
    ],
)(_stage_b_body)


def kernel(indices, table):
    tableT = table.T
    tail = jnp.pad(table[_LASTSB * _SBW:], ((0, 63), (0, 0))).T
    idx2d = indices.astype(jnp.int32).reshape(128, 128)
    et, bmap = _stage_a(tableT, tail, idx2d)
    out_pad = _stage_b(et, bmap)
    return out_pad[:_BATCH]
